# baseline (speedup 1.0000x reference)
import functools

import jax
import jax.numpy as jnp
from jax import lax
from jax.experimental import pallas as pl
from jax.experimental.pallas import tpu as pltpu
from jax.experimental.pallas import tpu_sc as plsc

D = 512
B = 8192
K = 17
LANE = 16

NC, NS = 2, 16
NW = NC * NS
EPW = B // NW
CHUNK = 4
NCH = EPW // CHUNK
ROWS = CHUNK * K
RPAD = 72
COLV = D // LANE
NBUF = 2

_mesh = plsc.VectorSubcoreMesh(
    core_axis_name="c", subcore_axis_name="s", num_cores=NC, num_subcores=NS
)


@functools.partial(
    pl.kernel,
    out_type=jax.ShapeDtypeStruct((B, D), jnp.float32),
    mesh=_mesh,
    scratch_types=[
        [pltpu.VMEM((RPAD,), jnp.int32) for _ in range(NBUF)],
        [pltpu.VMEM((RPAD, D), jnp.float32) for _ in range(NBUF)],
        pltpu.VMEM((CHUNK, D), jnp.float32),
        [pltpu.SemaphoreType.DMA for _ in range(NBUF)],
    ],
)
def _gather_sum(feat_hbm, idx_hbm, out_hbm, idx_bufs, rows_bufs, acc_v, sems):
    wid = lax.axis_index("s") * NC + lax.axis_index("c")
    cbase = wid * NCH

    def start_gather(c, b):
        pltpu.sync_copy(idx_hbm.at[pl.ds((cbase + c) * RPAD, RPAD)], idx_bufs[b])
        pltpu.async_copy(feat_hbm.at[idx_bufs[b]], rows_bufs[b], sems[b])

    for b in range(NBUF):
        start_gather(b, b)

    def super_body(s, carry):
        for b in range(NBUF):
            c = s * NBUF + b
            rows_v = rows_bufs[b]
            pltpu.make_async_copy(feat_hbm.at[idx_bufs[b]], rows_v, sems[b]).wait()

            def col_body(cv, c2):
                sl = pl.ds(cv * LANE, LANE)
                for j in range(CHUNK):
                    base = j * K
                    t = [rows_v[base + r, sl] for r in range(K)]
                    while len(t) > 1:
                        t = [t[i] + t[i + 1] for i in range(0, len(t) - 1, 2)] + (
                            [t[-1]] if len(t) % 2 else []
                        )
                    acc_v[j, sl] = t[0]
                return c2

            lax.fori_loop(0, COLV, col_body, 0)
            pltpu.sync_copy(acc_v, out_hbm.at[pl.ds(wid * EPW + c * CHUNK, CHUNK)])

            @pl.when(c + NBUF < NCH)
            def _start_next():
                start_gather(c + NBUF, b)

        return carry

    lax.fori_loop(0, NCH // NBUF, super_body, 0)


BM = 1024


def _mm_body(x_ref, w_ref, o_ref):
    y = jnp.dot(x_ref[...], w_ref[...], preferred_element_type=jnp.float32)
    o_ref[...] = jnp.maximum(y * (1.0 / K), 0.0)


def _matmul_relu(x, w):
    return pl.pallas_call(
        _mm_body,
        grid=(B // BM,),
        in_specs=[
            pl.BlockSpec((BM, D), lambda i: (i, 0)),
            pl.BlockSpec((D, D), lambda i: (0, 0)),
        ],
        out_specs=pl.BlockSpec((BM, D), lambda i: (i, 0)),
        out_shape=jax.ShapeDtypeStruct((B, D), jnp.float32),
    )(x, w)


def kernel(features, node, neighbours, neigh_weights):
    idx = jnp.concatenate([neighbours, node], axis=1).reshape(NW * NCH, ROWS)
    idx = jnp.pad(idx, ((0, 0), (0, RPAD - ROWS))).reshape(-1)
    sums = _gather_sum(features, idx)
    return _matmul_relu(sums, neigh_weights)


# speedup vs baseline: 1.0017x; 1.0017x over previous
import functools

import jax
import jax.numpy as jnp
from jax import lax
from jax.experimental import pallas as pl
from jax.experimental.pallas import tpu as pltpu
from jax.experimental.pallas import tpu_sc as plsc

D = 512
B = 8192
K = 17
LANE = 16

NC, NS = 2, 16
NW = NC * NS
EPW = B // NW
CHUNK = 4
NCH = EPW // CHUNK
ROWS = CHUNK * K
RPAD = 72
COLV = D // LANE
NBUF = 2

_mesh = plsc.VectorSubcoreMesh(
    core_axis_name="c", subcore_axis_name="s", num_cores=NC, num_subcores=NS
)


@functools.partial(
    pl.kernel,
    out_type=jax.ShapeDtypeStruct((B, D), jnp.float32),
    mesh=_mesh,
    scratch_types=[
        pltpu.VMEM((NCH * RPAD,), jnp.int32),
        [pltpu.VMEM((RPAD, D), jnp.float32) for _ in range(NBUF)],
        pltpu.VMEM((CHUNK, D), jnp.float32),
        [pltpu.SemaphoreType.DMA for _ in range(NBUF)],
    ],
)
def _gather_sum(feat_hbm, idx_hbm, out_hbm, idx_all, rows_bufs, acc_v, sems):
    wid = lax.axis_index("s") * NC + lax.axis_index("c")

    pltpu.sync_copy(idx_hbm.at[pl.ds(wid * NCH * RPAD, NCH * RPAD)], idx_all)

    def start_gather(c, b):
        idx_ref = idx_all.at[pl.ds(c * RPAD, RPAD)]
        pltpu.async_copy(feat_hbm.at[idx_ref], rows_bufs[b], sems[b])

    for b in range(NBUF):
        start_gather(b, b)

    def super_body(s, carry):
        for b in range(NBUF):
            c = s * NBUF + b
            rows_v = rows_bufs[b]
            idx_ref = idx_all.at[pl.ds(c * RPAD, RPAD)]
            pltpu.make_async_copy(feat_hbm.at[idx_ref], rows_v, sems[b]).wait()

            def col_body(cv, c2):
                sl = pl.ds(cv * LANE, LANE)
                for j in range(CHUNK):
                    base = j * K
                    t = [rows_v[base + r, sl] for r in range(K)]
                    while len(t) > 1:
                        t = [t[i] + t[i + 1] for i in range(0, len(t) - 1, 2)] + (
                            [t[-1]] if len(t) % 2 else []
                        )
                    acc_v[j, sl] = t[0]
                return c2

            lax.fori_loop(0, COLV, col_body, 0)
            pltpu.sync_copy(acc_v, out_hbm.at[pl.ds(wid * EPW + c * CHUNK, CHUNK)])

            @pl.when(c + NBUF < NCH)
            def _start_next():
                start_gather(c + NBUF, b)

        return carry

    lax.fori_loop(0, NCH // NBUF, super_body, 0)


BM = 1024


def _mm_body(x_ref, w_ref, o_ref):
    y = jnp.dot(x_ref[...], w_ref[...], preferred_element_type=jnp.float32)
    o_ref[...] = jnp.maximum(y * (1.0 / K), 0.0)


def _matmul_relu(x, w):
    return pl.pallas_call(
        _mm_body,
        grid=(B // BM,),
        in_specs=[
            pl.BlockSpec((BM, D), lambda i: (i, 0)),
            pl.BlockSpec((D, D), lambda i: (0, 0)),
        ],
        out_specs=pl.BlockSpec((BM, D), lambda i: (i, 0)),
        out_shape=jax.ShapeDtypeStruct((B, D), jnp.float32),
    )(x, w)


def kernel(features, node, neighbours, neigh_weights):
    idx = jnp.concatenate([neighbours, node], axis=1).reshape(NW * NCH, ROWS)
    idx = jnp.pad(idx, ((0, 0), (0, RPAD - ROWS))).reshape(-1)
    sums = _gather_sum(features, idx)
    return _matmul_relu(sums, neigh_weights)
